# baseline (speedup 1.0000x reference)
import functools

import jax
import jax.numpy as jnp
from jax import lax
from jax.experimental import pallas as pl
from jax.experimental.pallas import tpu as pltpu
from jax.experimental.pallas import tpu_sc as plsc

BATCH = 16384
DU = 64
DL = 32
H = 128
NC = 2
NS = 16
NW = NC * NS
B_PER_W = BATCH // NW

NUSERS = 1000000
CBLK = 16384
NCB = 16
OFF = NCB * CBLK
_LAST_BLK = (NUSERS - 1) // CBLK


def _bits16(x):
    b = lax.bitcast_convert_type(x.astype(jnp.bfloat16), jnp.uint16)
    return b.astype(jnp.uint32)


def _conv_body(q0_ref, q1_ref, q2_ref, q3_ref, out_ref):
    w01 = (_bits16(q0_ref[...]) << 16) | _bits16(q1_ref[...])
    w23 = (_bits16(q2_ref[...]) << 16) | _bits16(q3_ref[...])
    packed = jnp.concatenate([w01.T, w23.T], axis=1)
    out_ref[...] = lax.bitcast_convert_type(packed, jnp.float32)


def _tc_convert(tableT):
    return pl.pallas_call(
        _conv_body,
        grid=(NCB,),
        in_specs=[
            pl.BlockSpec((DU, CBLK), lambda i: (0, i)),
            pl.BlockSpec((DU, CBLK), lambda i: (0, i + NCB)),
            pl.BlockSpec((DU, CBLK), lambda i: (0, i + 2 * NCB)),
            pl.BlockSpec((DU, CBLK),
                         lambda i: (0, jnp.minimum(i + 3 * NCB, _LAST_BLK))),
        ],
        out_specs=pl.BlockSpec((CBLK, 2 * DU), lambda i: (i, 0)),
        out_shape=jax.ShapeDtypeStruct((OFF, 2 * DU), jnp.float32),
        compiler_params=pltpu.CompilerParams(
            vmem_limit_bytes=100 * 1024 * 1024),
    )(tableT, tableT, tableT, tableT)


@functools.lru_cache(maxsize=None)
def _make_sc_gather():
    mesh = plsc.VectorSubcoreMesh(
        core_axis_name="c", subcore_axis_name="s", num_cores=NC,
        num_subcores=NS,
    )

    @functools.partial(
        pl.kernel,
        out_type=jax.ShapeDtypeStruct((BATCH, DU), jnp.float32),
        mesh=mesh,
        scratch_types=[
            pltpu.VMEM((B_PER_W,), jnp.int32),
            pltpu.VMEM((B_PER_W,), jnp.int32),
            pltpu.VMEM((B_PER_W // 2, 2 * DU), jnp.float32),
            pltpu.VMEM((B_PER_W, DU), jnp.float32),
            pltpu.SemaphoreType.DMA,
        ],
        compiler_params=pltpu.CompilerParams(needs_layout_passes=False),
    )
    def _sc_gather(table_hbm, idx_hbm, qh_hbm, out_hbm, idx_v, qh_v, rows_v,
                   out_v, sem):
        wid = lax.axis_index("s") * NC + lax.axis_index("c")
        base = wid * B_PER_W
        half = B_PER_W // 2
        pltpu.sync_copy(idx_hbm.at[pl.ds(base, B_PER_W)], idx_v)
        pltpu.sync_copy(qh_hbm.at[pl.ds(base, B_PER_W)], qh_v)

        for h in range(2):
            hbase = h * half
            pltpu.async_copy(table_hbm.at[idx_v.at[pl.ds(hbase, half)]],
                             rows_v, sem).wait()

            def sel_grp(g, carry):
                ridx = g * 16 + lax.iota(jnp.int32, 16)
                qh = plsc.load_gather(qh_v, [hbase + ridx])
                cbase = qh * DU
                for w in range(DU):
                    vals = plsc.load_gather(rows_v, [ridx, cbase + w])
                    plsc.store_scatter(out_v,
                                       [hbase + ridx,
                                        jnp.full((16,), w, jnp.int32)],
                                       vals)
                return carry

            lax.fori_loop(0, half // 16, sel_grp, 0)

        pltpu.sync_copy(out_v, out_hbm.at[pl.ds(base, B_PER_W)])

    return _sc_gather


BLK = 2048


def _mlp_body(u2_ref, q_ref, lid_ref, stats_ref, lt_ref, w1_ref, b1_ref,
              w2_ref, b2_ref, out_ref):
    bits = lax.bitcast_convert_type(u2_ref[...], jnp.uint32)
    q = q_ref[...]
    vb = jnp.where((q & 1) == 0, bits & jnp.uint32(0xFFFF0000),
                   bits << 16)
    u = lax.bitcast_convert_type(vb, jnp.float32)
    acc = jnp.dot(u, w1_ref[0:DU, :], preferred_element_type=jnp.float32)

    lid = lid_ref[...]
    iota = lax.broadcasted_iota(jnp.int32, (BLK, H), 1)
    onehot = jnp.where(iota == lid, 1.0, 0.0)
    lp = jnp.dot(lt_ref[...], w1_ref[DU:DU + DL, :],
                 preferred_element_type=jnp.float32)
    acc = acc + jnp.dot(onehot, lp, preferred_element_type=jnp.float32)

    s = stats_ref[...]
    acc = acc + s[:, 0:1] * w1_ref[DU + DL:DU + DL + 1, :]
    acc = acc + s[:, 1:2] * w1_ref[DU + DL + 1:DU + DL + 2, :]

    h = jnp.maximum(acc + b1_ref[...], 0.0)
    out_ref[...] = (
        jnp.dot(h, w2_ref[...], preferred_element_type=jnp.float32)
        + b2_ref[...]
    )


def _tc_mlp(u2, par2, lid2, stats, lt_pad, W1, b1, W2, b2):
    nblk = BATCH // BLK
    grid = (nblk,)
    return pl.pallas_call(
        _mlp_body,
        grid=grid,
        in_specs=[
            pl.BlockSpec((BLK, DU), lambda i: (i, 0)),
            pl.BlockSpec((BLK, 1), lambda i: (i, 0)),
            pl.BlockSpec((BLK, 1), lambda i: (i, 0)),
            pl.BlockSpec((BLK, 2), lambda i: (i, 0)),
            pl.BlockSpec((H, DL), lambda i: (0, 0)),
            pl.BlockSpec((DU + DL + 2, H), lambda i: (0, 0)),
            pl.BlockSpec((1, H), lambda i: (0, 0)),
            pl.BlockSpec((H, H), lambda i: (0, 0)),
            pl.BlockSpec((1, H), lambda i: (0, 0)),
        ],
        out_specs=pl.BlockSpec((BLK, H), lambda i: (i, 0)),
        out_shape=jax.ShapeDtypeStruct((BATCH, H), jnp.float32),
    )(u2, par2, lid2, stats, lt_pad, W1, b1, W2, b2)


def kernel(user_ids, lang_ids, stats, user_table, lang_table, W1, b1, W2, b2):
    ids = user_ids.astype(jnp.int32)
    packed = _tc_convert(user_table.T)
    q = ids // OFF
    row = ids - q * OFF
    u2 = _make_sc_gather()(packed, row, q >> 1)
    par2 = q.reshape(-1, 1)
    lt_pad = jnp.pad(lang_table, ((0, H - lang_table.shape[0]), (0, 0)))
    lid2 = lang_ids.astype(jnp.int32).reshape(-1, 1)
    return _tc_mlp(u2, par2, lid2, stats, lt_pad, W1, b1.reshape(1, -1), W2,
                   b2.reshape(1, -1))


# speedup vs baseline: 1.1429x; 1.1429x over previous
import functools

import jax
import jax.numpy as jnp
from jax import lax
from jax.experimental import pallas as pl
from jax.experimental.pallas import tpu as pltpu
from jax.experimental.pallas import tpu_sc as plsc

BATCH = 16384
DU = 64
DL = 32
H = 128
NC = 2
NS = 16
NW = NC * NS
B_PER_W = BATCH // NW

NUSERS = 1000000
CBLK = 16384
NCB = 16
OFF = NCB * CBLK
_LAST_BLK = (NUSERS - 1) // CBLK


def _bits16(x):
    b = lax.bitcast_convert_type(x.astype(jnp.bfloat16), jnp.uint16)
    return b.astype(jnp.uint32)


def _conv_body(q0_ref, q1_ref, q2_ref, q3_ref, out_ref):
    w01 = (_bits16(q0_ref[...]) << 16) | _bits16(q1_ref[...])
    w23 = (_bits16(q2_ref[...]) << 16) | _bits16(q3_ref[...])
    packed = jnp.concatenate([w01.T, w23.T], axis=1)
    out_ref[...] = lax.bitcast_convert_type(packed, jnp.float32)


def _tc_convert(tableT):
    return pl.pallas_call(
        _conv_body,
        grid=(NCB,),
        in_specs=[
            pl.BlockSpec((DU, CBLK), lambda i: (0, i)),
            pl.BlockSpec((DU, CBLK), lambda i: (0, i + NCB)),
            pl.BlockSpec((DU, CBLK), lambda i: (0, i + 2 * NCB)),
            pl.BlockSpec((DU, CBLK),
                         lambda i: (0, jnp.minimum(i + 3 * NCB, _LAST_BLK))),
        ],
        out_specs=pl.BlockSpec((CBLK, 2 * DU), lambda i: (i, 0)),
        out_shape=jax.ShapeDtypeStruct((OFF, 2 * DU), jnp.float32),
        compiler_params=pltpu.CompilerParams(
            vmem_limit_bytes=100 * 1024 * 1024),
    )(tableT, tableT, tableT, tableT)


@functools.lru_cache(maxsize=None)
def _make_sc_gather():
    mesh = plsc.VectorSubcoreMesh(
        core_axis_name="c", subcore_axis_name="s", num_cores=NC,
        num_subcores=NS,
    )

    @functools.partial(
        pl.kernel,
        out_type=jax.ShapeDtypeStruct((BATCH, 2 * DU), jnp.float32),
        mesh=mesh,
        scratch_types=[
            pltpu.VMEM((B_PER_W,), jnp.int32),
            pltpu.VMEM((B_PER_W, 2 * DU), jnp.float32),
            pltpu.SemaphoreType.DMA,
        ],
    )
    def _sc_gather(table_hbm, idx_hbm, out_hbm, idx_v, rows_v, sem):
        wid = lax.axis_index("s") * NC + lax.axis_index("c")
        base = wid * B_PER_W
        pltpu.sync_copy(idx_hbm.at[pl.ds(base, B_PER_W)], idx_v)
        pltpu.async_copy(table_hbm.at[idx_v], rows_v, sem).wait()
        pltpu.sync_copy(rows_v, out_hbm.at[pl.ds(base, B_PER_W)])

    return _sc_gather


BLK = 2048


def _mlp_body(u2_ref, q_ref, lid_ref, stats_ref, lt_ref, w1_ref, b1_ref,
              w2_ref, b2_ref, out_ref):
    bits = lax.bitcast_convert_type(u2_ref[...], jnp.uint32)
    q = q_ref[...]
    halfw = jnp.where(q >= 2, bits[:, DU:2 * DU], bits[:, 0:DU])
    vb = jnp.where((q & 1) == 0, halfw & jnp.uint32(0xFFFF0000),
                   halfw << 16)
    u = lax.bitcast_convert_type(vb, jnp.float32)
    acc = jnp.dot(u, w1_ref[0:DU, :], preferred_element_type=jnp.float32)

    lid = lid_ref[...]
    iota = lax.broadcasted_iota(jnp.int32, (BLK, H), 1)
    onehot = jnp.where(iota == lid, 1.0, 0.0)
    lp = jnp.dot(lt_ref[...], w1_ref[DU:DU + DL, :],
                 preferred_element_type=jnp.float32)
    acc = acc + jnp.dot(onehot, lp, preferred_element_type=jnp.float32)

    s = stats_ref[...]
    acc = acc + s[:, 0:1] * w1_ref[DU + DL:DU + DL + 1, :]
    acc = acc + s[:, 1:2] * w1_ref[DU + DL + 1:DU + DL + 2, :]

    h = jnp.maximum(acc + b1_ref[...], 0.0)
    out_ref[...] = (
        jnp.dot(h, w2_ref[...], preferred_element_type=jnp.float32)
        + b2_ref[...]
    )


def _tc_mlp(u2, par2, lid2, stats, lt_pad, W1, b1, W2, b2):
    nblk = BATCH // BLK
    grid = (nblk,)
    return pl.pallas_call(
        _mlp_body,
        grid=grid,
        in_specs=[
            pl.BlockSpec((BLK, 2 * DU), lambda i: (i, 0)),
            pl.BlockSpec((BLK, 1), lambda i: (i, 0)),
            pl.BlockSpec((BLK, 1), lambda i: (i, 0)),
            pl.BlockSpec((BLK, 2), lambda i: (i, 0)),
            pl.BlockSpec((H, DL), lambda i: (0, 0)),
            pl.BlockSpec((DU + DL + 2, H), lambda i: (0, 0)),
            pl.BlockSpec((1, H), lambda i: (0, 0)),
            pl.BlockSpec((H, H), lambda i: (0, 0)),
            pl.BlockSpec((1, H), lambda i: (0, 0)),
        ],
        out_specs=pl.BlockSpec((BLK, H), lambda i: (i, 0)),
        out_shape=jax.ShapeDtypeStruct((BATCH, H), jnp.float32),
    )(u2, par2, lid2, stats, lt_pad, W1, b1, W2, b2)


def kernel(user_ids, lang_ids, stats, user_table, lang_table, W1, b1, W2, b2):
    ids = user_ids.astype(jnp.int32)
    packed = _tc_convert(user_table.T)
    q = ids // OFF
    row = ids - q * OFF
    u2 = _make_sc_gather()(packed, row)
    par2 = q.reshape(-1, 1)
    lt_pad = jnp.pad(lang_table, ((0, H - lang_table.shape[0]), (0, 0)))
    lid2 = lang_ids.astype(jnp.int32).reshape(-1, 1)
    return _tc_mlp(u2, par2, lid2, stats, lt_pad, W1, b1.reshape(1, -1), W2,
                   b2.reshape(1, -1))
